# Initial kernel scaffold; baseline (speedup 1.0000x reference)
import jax
import jax.numpy as jnp
from jax.experimental import pallas as pl


def kernel(x, edge_index, batch, eFeature, params):
    raise NotImplementedError("write your pallas kernel here")


# speedup vs baseline: 1.7561x; 1.7561x over previous
import jax
import jax.numpy as jnp
from jax.experimental import pallas as pl
from jax.experimental.pallas import tpu as pltpu


def _mm_kernel(a_ref, b_ref, o_ref):
    @pl.when(pl.program_id(2) == 0)
    def _():
        o_ref[...] = jnp.zeros_like(o_ref)

    o_ref[...] += jnp.dot(a_ref[...], b_ref[...],
                          preferred_element_type=jnp.float32)


def _ceil_to(v, m):
    return (v + m - 1) // m * m


def _mm(a, b, bm=256, bn=256, bk=512):
    M, K = a.shape
    K2, Nc = b.shape
    assert K == K2, (a.shape, b.shape)
    bm = min(bm, _ceil_to(M, 8))
    bn = min(bn, _ceil_to(Nc, 128))
    bk = min(bk, _ceil_to(K, 128))
    Mp, Kp, Np = _ceil_to(M, bm), _ceil_to(K, bk), _ceil_to(Nc, bn)
    if (Mp, Kp) != (M, K):
        a = jnp.pad(a, ((0, Mp - M), (0, Kp - K)))
    if (Kp, Np) != (K, Nc):
        b = jnp.pad(b, ((0, Kp - K), (0, Np - Nc)))
    out = pl.pallas_call(
        _mm_kernel,
        grid=(Mp // bm, Np // bn, Kp // bk),
        in_specs=[
            pl.BlockSpec((bm, bk), lambda i, j, k: (i, k)),
            pl.BlockSpec((bk, bn), lambda i, j, k: (k, j)),
        ],
        out_specs=pl.BlockSpec((bm, bn), lambda i, j, k: (i, j)),
        out_shape=jax.ShapeDtypeStruct((Mp, Np), jnp.float32),
        compiler_params=pltpu.CompilerParams(
            dimension_semantics=("parallel", "parallel", "arbitrary")),
    )(a, b)
    if (Mp, Np) != (M, Nc):
        out = out[:M, :Nc]
    return out


def kernel(x, edge_index, batch, eFeature, params):
    n = x.shape[0]
    ng = eFeature.shape[0]
    n_pad = _ceil_to(n, 512)

    src = edge_index[0]
    dst = edge_index[1]
    adj = jnp.zeros((n_pad, n_pad), jnp.float32).at[dst, src].add(1.0)

    def agg_mm(rhs):
        rhs = jnp.pad(rhs, ((0, n_pad - n), (0, 0)))
        return _mm(adj, rhs)[:n]

    h = x
    deg = None
    for i in range(7):
        hn = _mm(h, params[f"conv{i+1}_Wn"])
        if deg is None:
            ones = jnp.ones((n, 1), jnp.float32)
            aggd = agg_mm(jnp.concatenate([hn, ones], axis=1))
            agg = aggd[:, :-1]
            deg = jnp.clip(aggd[:, -1], 1.0)
        else:
            agg = agg_mm(hn)
        z = agg / deg[:, None] + _mm(h, params[f"conv{i+1}_Wr"])
        z = z + params[f"conv{i+1}_b"]
        mu = z.mean(axis=0)
        var = z.var(axis=0)
        z = (z - mu) / jnp.sqrt(var + 1e-5)
        z = z * params[f"bn{i+1}_g"] + params[f"bn{i+1}_b"]
        z = jax.nn.elu(z)
        skip = _mm(h, params[f"skip{i+1}_W"]) + params[f"skip{i+1}_b"]
        h = jnp.concatenate([skip, z], axis=1)
        if i == 2:
            h = jax.nn.elu(_mm(h, params["red_W"]))

    groups = jnp.arange(ng, dtype=batch.dtype)
    pmat = (batch[None, :] == groups[:, None]).astype(jnp.float32)
    counts = jnp.clip(pmat.sum(axis=1), 1.0)
    hg = _mm(pmat, h) / counts[:, None]
    hg = jnp.concatenate([hg, eFeature], axis=1)
    hg = jax.nn.elu(_mm(hg, params["fc1_W"]) + params["fc1_b"])
    return _mm(hg, params["fc3_W"]) + params["fc3_b"]


# speedup vs baseline: 1.9259x; 1.0967x over previous
import jax
import jax.numpy as jnp
from jax.experimental import pallas as pl
from jax.experimental.pallas import tpu as pltpu


def _mm_kernel(a_ref, b_ref, o_ref):
    @pl.when(pl.program_id(2) == 0)
    def _():
        o_ref[...] = jnp.zeros_like(o_ref)

    o_ref[...] += jnp.dot(a_ref[...], b_ref[...],
                          preferred_element_type=jnp.float32)


def _ceil_to(v, m):
    return (v + m - 1) // m * m


def _mm(a, b, bm=256, bn=256, bk=512, cast_bf16=False):
    if cast_bf16:
        a = a.astype(jnp.bfloat16)
        b = b.astype(jnp.bfloat16)
    M, K = a.shape
    K2, Nc = b.shape
    assert K == K2, (a.shape, b.shape)
    bm = min(bm, _ceil_to(M, 8))
    bn = min(bn, _ceil_to(Nc, 128))
    bk = min(bk, _ceil_to(K, 128))
    Mp, Kp, Np = _ceil_to(M, bm), _ceil_to(K, bk), _ceil_to(Nc, bn)
    if (Mp, Kp) != (M, K):
        a = jnp.pad(a, ((0, Mp - M), (0, Kp - K)))
    if (Kp, Np) != (K, Nc):
        b = jnp.pad(b, ((0, Kp - K), (0, Np - Nc)))
    out = pl.pallas_call(
        _mm_kernel,
        grid=(Mp // bm, Np // bn, Kp // bk),
        in_specs=[
            pl.BlockSpec((bm, bk), lambda i, j, k: (i, k)),
            pl.BlockSpec((bk, bn), lambda i, j, k: (k, j)),
        ],
        out_specs=pl.BlockSpec((bm, bn), lambda i, j, k: (i, j)),
        out_shape=jax.ShapeDtypeStruct((Mp, Np), jnp.float32),
        compiler_params=pltpu.CompilerParams(
            dimension_semantics=("parallel", "parallel", "arbitrary")),
    )(a, b)
    if (Mp, Np) != (M, Nc):
        out = out[:M, :Nc]
    return out


def kernel(x, edge_index, batch, eFeature, params):
    n = x.shape[0]
    ng = eFeature.shape[0]
    n_pad = _ceil_to(n, 512)

    src = edge_index[0]
    dst = edge_index[1]
    adj = jnp.zeros((n_pad, n_pad), jnp.float32).at[dst, src].add(1.0)

    adj_bf = adj.astype(jnp.bfloat16)

    def agg_mm(rhs):
        rhs = jnp.pad(rhs, ((0, n_pad - n), (0, 0)))
        return _mm(adj_bf, rhs, cast_bf16=True)[:n]

    h = x
    deg = None
    for i in range(7):
        hn = _mm(h, params[f"conv{i+1}_Wn"])
        if deg is None:
            ones = jnp.ones((n, 1), jnp.float32)
            aggd = agg_mm(jnp.concatenate([hn, ones], axis=1))
            agg = aggd[:, :-1]
            deg = jnp.clip(aggd[:, -1], 1.0)
        else:
            agg = agg_mm(hn)
        z = agg / deg[:, None] + _mm(h, params[f"conv{i+1}_Wr"])
        z = z + params[f"conv{i+1}_b"]
        mu = z.mean(axis=0)
        var = z.var(axis=0)
        z = (z - mu) / jnp.sqrt(var + 1e-5)
        z = z * params[f"bn{i+1}_g"] + params[f"bn{i+1}_b"]
        z = jax.nn.elu(z)
        skip = _mm(h, params[f"skip{i+1}_W"]) + params[f"skip{i+1}_b"]
        h = jnp.concatenate([skip, z], axis=1)
        if i == 2:
            h = jax.nn.elu(_mm(h, params["red_W"]))

    groups = jnp.arange(ng, dtype=batch.dtype)
    pmat = (batch[None, :] == groups[:, None]).astype(jnp.float32)
    counts = jnp.clip(pmat.sum(axis=1), 1.0)
    hg = _mm(pmat, h) / counts[:, None]
    hg = jnp.concatenate([hg, eFeature], axis=1)
    hg = jax.nn.elu(_mm(hg, params["fc1_W"]) + params["fc1_b"])
    return _mm(hg, params["fc3_W"]) + params["fc3_b"]


# speedup vs baseline: 1.9919x; 1.0343x over previous
import jax
import jax.numpy as jnp
from jax.experimental import pallas as pl
from jax.experimental.pallas import tpu as pltpu


def _mm_kernel(a_ref, b_ref, o_ref):
    @pl.when(pl.program_id(2) == 0)
    def _():
        o_ref[...] = jnp.zeros_like(o_ref)

    o_ref[...] += jnp.dot(a_ref[...], b_ref[...],
                          preferred_element_type=jnp.float32)


def _ceil_to(v, m):
    return (v + m - 1) // m * m


def _mm(a, b, bm=256, bn=256, bk=512, cast_bf16=False):
    if cast_bf16:
        a = a.astype(jnp.bfloat16)
        b = b.astype(jnp.bfloat16)
    M, K = a.shape
    K2, Nc = b.shape
    assert K == K2, (a.shape, b.shape)
    bm = min(bm, _ceil_to(M, 8))
    bn = min(bn, _ceil_to(Nc, 128))
    bk = min(bk, _ceil_to(K, 128))
    Mp, Kp, Np = _ceil_to(M, bm), _ceil_to(K, bk), _ceil_to(Nc, bn)
    if (Mp, Kp) != (M, K):
        a = jnp.pad(a, ((0, Mp - M), (0, Kp - K)))
    if (Kp, Np) != (K, Nc):
        b = jnp.pad(b, ((0, Kp - K), (0, Np - Nc)))
    out = pl.pallas_call(
        _mm_kernel,
        grid=(Mp // bm, Np // bn, Kp // bk),
        in_specs=[
            pl.BlockSpec((bm, bk), lambda i, j, k: (i, k)),
            pl.BlockSpec((bk, bn), lambda i, j, k: (k, j)),
        ],
        out_specs=pl.BlockSpec((bm, bn), lambda i, j, k: (i, j)),
        out_shape=jax.ShapeDtypeStruct((Mp, Np), jnp.float32),
        compiler_params=pltpu.CompilerParams(
            dimension_semantics=("parallel", "parallel", "arbitrary")),
    )(a, b)
    if (Mp, Np) != (M, Nc):
        out = out[:M, :Nc]
    return out


def kernel(x, edge_index, batch, eFeature, params):
    n = x.shape[0]
    ng = eFeature.shape[0]
    n_pad = _ceil_to(n, 512)

    src = edge_index[0]
    dst = edge_index[1]
    adj = jnp.zeros((n_pad, n_pad), jnp.float32).at[dst, src].add(1.0)

    adj_bf = adj.astype(jnp.bfloat16)

    def agg_mm(rhs):
        rhs = jnp.pad(rhs, ((0, n_pad - n), (0, 0)))
        return _mm(adj_bf, rhs, cast_bf16=True)[:n]

    h = x
    deg = None
    for i in range(7):
        do = params[f"conv{i+1}_Wn"].shape[1]
        w_all = jnp.concatenate([params[f"conv{i+1}_Wn"],
                                 params[f"conv{i+1}_Wr"],
                                 params[f"skip{i+1}_W"]], axis=1)
        hall = _mm(h, w_all, cast_bf16=True)
        hn, hr, skip = hall[:, :do], hall[:, do:2 * do], hall[:, 2 * do:]
        if deg is None:
            ones = jnp.ones((n, 1), jnp.float32)
            aggd = agg_mm(jnp.concatenate([hn, ones], axis=1))
            agg = aggd[:, :-1]
            deg = jnp.clip(aggd[:, -1], 1.0)
        else:
            agg = agg_mm(hn)
        z = agg / deg[:, None] + hr
        z = z + params[f"conv{i+1}_b"]
        mu = z.mean(axis=0)
        var = z.var(axis=0)
        z = (z - mu) / jnp.sqrt(var + 1e-5)
        z = z * params[f"bn{i+1}_g"] + params[f"bn{i+1}_b"]
        z = jax.nn.elu(z)
        h = jnp.concatenate([skip + params[f"skip{i+1}_b"], z], axis=1)
        if i == 2:
            h = jax.nn.elu(_mm(h, params["red_W"], cast_bf16=True))

    groups = jnp.arange(ng, dtype=batch.dtype)
    pmat = (batch[None, :] == groups[:, None]).astype(jnp.float32)
    counts = jnp.clip(pmat.sum(axis=1), 1.0)
    hg = _mm(pmat, h, cast_bf16=True) / counts[:, None]
    hg = jnp.concatenate([hg, eFeature], axis=1)
    hg = jax.nn.elu(_mm(hg, params["fc1_W"], cast_bf16=True) + params["fc1_b"])
    return _mm(hg, params["fc3_W"], cast_bf16=True) + params["fc3_b"]


# speedup vs baseline: 4.8377x; 2.4287x over previous
import jax
import jax.numpy as jnp
from jax.experimental import pallas as pl
from jax.experimental.pallas import tpu as pltpu


def _mm_kernel(a_ref, b_ref, o_ref):
    @pl.when(pl.program_id(2) == 0)
    def _():
        o_ref[...] = jnp.zeros_like(o_ref)

    o_ref[...] += jnp.dot(a_ref[...], b_ref[...],
                          preferred_element_type=jnp.float32)


def _ceil_to(v, m):
    return (v + m - 1) // m * m


def _mm(a, b, bm=512, bn=512, bk=1024, cast_bf16=False):
    if cast_bf16:
        a = a.astype(jnp.bfloat16)
        b = b.astype(jnp.bfloat16)
    M, K = a.shape
    K2, Nc = b.shape
    assert K == K2, (a.shape, b.shape)
    bm = min(bm, _ceil_to(M, 8))
    bn = min(bn, _ceil_to(Nc, 128))
    bk = min(bk, _ceil_to(K, 128))
    Mp, Kp, Np = _ceil_to(M, bm), _ceil_to(K, bk), _ceil_to(Nc, bn)
    if (Mp, Kp) != (M, K):
        a = jnp.pad(a, ((0, Mp - M), (0, Kp - K)))
    if (Kp, Np) != (K, Nc):
        b = jnp.pad(b, ((0, Kp - K), (0, Np - Nc)))
    out = pl.pallas_call(
        _mm_kernel,
        grid=(Mp // bm, Np // bn, Kp // bk),
        in_specs=[
            pl.BlockSpec((bm, bk), lambda i, j, k: (i, k)),
            pl.BlockSpec((bk, bn), lambda i, j, k: (k, j)),
        ],
        out_specs=pl.BlockSpec((bm, bn), lambda i, j, k: (i, j)),
        out_shape=jax.ShapeDtypeStruct((Mp, Np), jnp.float32),
        compiler_params=pltpu.CompilerParams(
            dimension_semantics=("parallel", "parallel", "arbitrary")),
    )(a, b)
    if (Mp, Np) != (M, Nc):
        out = out[:M, :Nc]
    return out


def kernel(x, edge_index, batch, eFeature, params):
    n = x.shape[0]
    ng = eFeature.shape[0]
    n_pad = _ceil_to(n, 512)

    src = edge_index[0]
    dst = edge_index[1]
    adj = jnp.zeros((n_pad, n_pad), jnp.float32).at[dst, src].add(1.0)

    adj_bf = adj.astype(jnp.bfloat16)

    def agg_mm(rhs):
        rhs = jnp.pad(rhs, ((0, n_pad - n), (0, 0)))
        return _mm(adj_bf, rhs, bn=1024, cast_bf16=True)[:n]

    h = x
    deg = None
    for i in range(7):
        do = params[f"conv{i+1}_Wn"].shape[1]
        w_all = jnp.concatenate([params[f"conv{i+1}_Wn"],
                                 params[f"conv{i+1}_Wr"],
                                 params[f"skip{i+1}_W"]], axis=1)
        hall = _mm(h, w_all, cast_bf16=True)
        hn, hr, skip = hall[:, :do], hall[:, do:2 * do], hall[:, 2 * do:]
        if deg is None:
            ones = jnp.ones((n, 1), jnp.float32)
            aggd = agg_mm(jnp.concatenate([hn, ones], axis=1))
            agg = aggd[:, :-1]
            deg = jnp.clip(aggd[:, -1], 1.0)
        else:
            agg = agg_mm(hn)
        z = agg / deg[:, None] + hr
        z = z + params[f"conv{i+1}_b"]
        mu = z.mean(axis=0)
        var = z.var(axis=0)
        z = (z - mu) / jnp.sqrt(var + 1e-5)
        z = z * params[f"bn{i+1}_g"] + params[f"bn{i+1}_b"]
        z = jax.nn.elu(z)
        h = jnp.concatenate([skip + params[f"skip{i+1}_b"], z], axis=1)
        if i == 2:
            h = jax.nn.elu(_mm(h, params["red_W"], cast_bf16=True))

    groups = jnp.arange(ng, dtype=batch.dtype)
    pmat = (batch[None, :] == groups[:, None]).astype(jnp.float32)
    counts = jnp.clip(pmat.sum(axis=1), 1.0)
    hg = _mm(pmat, h, cast_bf16=True) / counts[:, None]
    hg = jnp.concatenate([hg, eFeature], axis=1)
    hg = jax.nn.elu(_mm(hg, params["fc1_W"], cast_bf16=True) + params["fc1_b"])
    return _mm(hg, params["fc3_W"], cast_bf16=True) + params["fc3_b"]
